# baseline (speedup 1.0000x reference)
import functools

import jax
import jax.numpy as jnp
from jax import lax
from jax.experimental import pallas as pl
from jax.experimental.pallas import tpu as pltpu
from jax.experimental.pallas import tpu_sc as plsc

DIM = 32
B = 16384
L = 20
NEG = 5

NC = 2
NS = 16
NW = NC * NS
BPW = B // NW
CB = 64
NCHUNK = BPW // CB
NGRP = CB // 16

CTX_SL = CB * L // 128
NEG_SL = CB * NEG // 64


def _sc_body(ctx_i, cen_i, neg_i, ctab, xtab,
             out_w, out_c, out_n,
             ci_v, ce_v, ni_v, ctx_buf, cen_buf, neg_buf,
             ct, wbuf, ob, sem):
    cid = lax.axis_index("c")
    sid = lax.axis_index("s")
    wid = sid * NC + cid
    iota16 = lax.iota(jnp.int32, 16)

    pltpu.sync_copy(ctx_i.at[pl.ds(wid * (BPW * L // 128), BPW * L // 128)], ci_v)
    pltpu.sync_copy(cen_i.at[pl.ds(wid * NCHUNK, NCHUNK)], ce_v)
    pltpu.sync_copy(neg_i.at[pl.ds(wid * (BPW * NEG // 64), BPW * NEG // 64)], ni_v)

    ctx_rows = ctx_buf
    cen_rows = cen_buf
    ob_rows = ob

    def chunk(k, carry):
        base = wid * BPW + k * CB
        cps = []
        for j in range(CTX_SL):
            cps.append(pltpu.async_copy(
                ctab.at[ci_v.at[k * CTX_SL + j]], ctx_rows.at[pl.ds(j * 128, 128)], sem))
        cps.append(pltpu.async_copy(xtab.at[ce_v.at[k]], cen_rows, sem))
        for j in range(NEG_SL):
            cps.append(pltpu.async_copy(
                xtab.at[ni_v.at[k * NEG_SL + j]], neg_buf.at[pl.ds(j * 64, 64)], sem))
        for cp in cps:
            cp.wait()
        pltpu.sync_copy(cen_rows, out_c.at[pl.ds(base, CB)])
        pltpu.sync_copy(neg_buf, out_n.at[pl.ds(base * NEG, CB * NEG)])

        for g in range(NGRP):
            bvec = iota16 + (g * 16)
            for d in range(DIM):
                dsp = jnp.full((16,), d, jnp.int32)
                ct[d, :] = plsc.load_gather(cen_buf, [bvec, dsp])
            rowv = bvec * L

            def lbody(l, wsum):
                row = rowv + l
                dist = jnp.zeros((16,), jnp.float32)
                for d in range(DIM):
                    dsp = jnp.full((16,), d, jnp.int32)
                    x = plsc.load_gather(ctx_buf, [row, dsp])
                    diff = x - ct[d, :]
                    dist = dist + diff * diff
                w = jnp.exp(dist * -0.5)
                wbuf[l, pl.ds(g * 16, 16)] = w
                return wsum + w

            wsum = lax.fori_loop(0, L, lbody, jnp.zeros((16,), jnp.float32))
            inv = 1.0 / (wsum + 1e-8)

            def dbody(d, carry2):
                dsp = jnp.full((16,), 1, jnp.int32) * d
                acc = jnp.zeros((16,), jnp.float32)
                for l in range(L):
                    x = plsc.load_gather(ctx_buf, [rowv + l, dsp])
                    acc = acc + wbuf[l, pl.ds(g * 16, 16)] * x
                plsc.store_scatter(ob, [bvec, dsp], acc * inv)
                return carry2

            lax.fori_loop(0, DIM, dbody, 0)
        pltpu.sync_copy(ob_rows, out_w.at[pl.ds(base, CB)])
        return carry

    lax.fori_loop(0, NCHUNK, chunk, 0)


@jax.jit
def _run(ctx_i, cen_i, neg_i, ctab, xtab):
    mesh = plsc.VectorSubcoreMesh(core_axis_name="c", subcore_axis_name="s")
    f = pl.kernel(
        _sc_body,
        out_type=(
            jax.ShapeDtypeStruct((B, DIM), jnp.float32),
            jax.ShapeDtypeStruct((B, DIM), jnp.float32),
            jax.ShapeDtypeStruct((B * NEG, DIM), jnp.float32),
        ),
        mesh=mesh,
        scratch_types=[
            pltpu.VMEM((BPW * L // 128, 128), jnp.int32),
            pltpu.VMEM((NCHUNK, 64), jnp.int32),
            pltpu.VMEM((BPW * NEG // 64, 64), jnp.int32),
            pltpu.VMEM((CB * L, DIM), jnp.float32),
            pltpu.VMEM((CB, DIM), jnp.float32),
            pltpu.VMEM((CB * NEG, DIM), jnp.float32),
            pltpu.VMEM((DIM, 16), jnp.float32),
            pltpu.VMEM((L, CB), jnp.float32),
            pltpu.VMEM((CB, DIM), jnp.float32),
            pltpu.SemaphoreType.DMA,
        ],
        compiler_params=pltpu.CompilerParams(
            needs_layout_passes=False, use_tc_tiling_on_sc=False),
    )
    return f(ctx_i, cen_i, neg_i, ctab, xtab)


def kernel(context, center, neg_samples, center_table, context_table):
    ctx_i = context.astype(jnp.int32).reshape(B * L // 128, 128)
    cen_i = center.astype(jnp.int32).reshape(B // 64, 64)
    neg_i = neg_samples.astype(jnp.int32).reshape(B * NEG // 64, 64)
    out_w, out_c, out_n = _run(ctx_i, cen_i, neg_i, center_table, context_table)
    return (out_w, out_c, out_n.reshape(B, NEG, DIM))


# speedup vs baseline: 1.3220x; 1.3220x over previous
import functools

import jax
import jax.numpy as jnp
from jax import lax
from jax.experimental import pallas as pl
from jax.experimental.pallas import tpu as pltpu
from jax.experimental.pallas import tpu_sc as plsc

DIM = 32
B = 16384
L = 20
NEG = 5

NC = 2
NS = 16
NW = NC * NS
BPW = B // NW
CB = 64
NCHUNK = BPW // CB
NGRP = CB // 16

CTX_SL = CB * L // 128
NEG_SL = CB * NEG // 64


def _sc_body(ctx_i, cen_i, neg_i, ctab, xtab,
             out_w, out_c, out_n,
             ci_v, ce_v, ni_v, ctx_buf, cen_buf, neg_buf, ob, sem):
    cid = lax.axis_index("c")
    sid = lax.axis_index("s")
    wid = sid * NC + cid

    pltpu.sync_copy(ctx_i.at[pl.ds(wid * (BPW * L // 128), BPW * L // 128)], ci_v)
    pltpu.sync_copy(cen_i.at[pl.ds(wid * NCHUNK, NCHUNK)], ce_v)
    pltpu.sync_copy(neg_i.at[pl.ds(wid * (BPW * NEG // 64), BPW * NEG // 64)], ni_v)

    ctx_rows = ctx_buf
    cen_rows = cen_buf
    ob_rows = ob

    def chunk(k, carry):
        base = wid * BPW + k * CB
        cps = []
        for j in range(CTX_SL):
            cps.append(pltpu.async_copy(
                ctab.at[ci_v.at[k * CTX_SL + j]], ctx_rows.at[pl.ds(j * 128, 128)], sem))
        cps.append(pltpu.async_copy(xtab.at[ce_v.at[k]], cen_rows, sem))
        for j in range(NEG_SL):
            cps.append(pltpu.async_copy(
                xtab.at[ni_v.at[k * NEG_SL + j]], neg_buf.at[pl.ds(j * 64, 64)], sem))
        for cp in cps:
            cp.wait()
        pltpu.sync_copy(cen_rows, out_c.at[pl.ds(base, CB)])
        pltpu.sync_copy(neg_buf, out_n.at[pl.ds(base * NEG, CB * NEG)])

        lane15 = jnp.full((16,), 15, jnp.int32)

        def bbody(b, carry2):
            c0 = cen_rows[b, pl.ds(0, 16)]
            c1 = cen_rows[b, pl.ds(16, 16)]
            acc0 = jnp.zeros((16,), jnp.float32)
            acc1 = jnp.zeros((16,), jnp.float32)
            wsum = jnp.zeros((16,), jnp.float32)
            row0 = b * L
            for l in range(L):
                x0 = ctx_rows[row0 + l, pl.ds(0, 16)]
                x1 = ctx_rows[row0 + l, pl.ds(16, 16)]
                d0 = x0 - c0
                d1 = x1 - c1
                s2 = d0 * d0 + d1 * d1
                tot = jnp.sum(s2)
                w = jnp.exp(jnp.broadcast_to(tot * -0.5, (16,)))
                acc0 = acc0 + w * x0
                acc1 = acc1 + w * x1
                wsum = wsum + w
            inv = 1.0 / (wsum + 1e-8)
            ob_rows[b, pl.ds(0, 16)] = acc0 * inv
            ob_rows[b, pl.ds(16, 16)] = acc1 * inv
            return carry2

        lax.fori_loop(0, CB, bbody, 0)
        pltpu.sync_copy(ob_rows, out_w.at[pl.ds(base, CB)])
        return carry

    lax.fori_loop(0, NCHUNK, chunk, 0)


@jax.jit
def _run(ctx_i, cen_i, neg_i, ctab, xtab):
    mesh = plsc.VectorSubcoreMesh(core_axis_name="c", subcore_axis_name="s")
    f = pl.kernel(
        _sc_body,
        out_type=(
            jax.ShapeDtypeStruct((B, DIM), jnp.float32),
            jax.ShapeDtypeStruct((B, DIM), jnp.float32),
            jax.ShapeDtypeStruct((B * NEG, DIM), jnp.float32),
        ),
        mesh=mesh,
        scratch_types=[
            pltpu.VMEM((BPW * L // 128, 128), jnp.int32),
            pltpu.VMEM((NCHUNK, 64), jnp.int32),
            pltpu.VMEM((BPW * NEG // 64, 64), jnp.int32),
            pltpu.VMEM((CB * L, DIM), jnp.float32),
            pltpu.VMEM((CB, DIM), jnp.float32),
            pltpu.VMEM((CB * NEG, DIM), jnp.float32),
            pltpu.VMEM((CB, DIM), jnp.float32),
            pltpu.SemaphoreType.DMA,
        ],
        compiler_params=pltpu.CompilerParams(
            needs_layout_passes=False, use_tc_tiling_on_sc=False),
    )
    return f(ctx_i, cen_i, neg_i, ctab, xtab)


def kernel(context, center, neg_samples, center_table, context_table):
    ctx_i = context.astype(jnp.int32).reshape(B * L // 128, 128)
    cen_i = center.astype(jnp.int32).reshape(B // 64, 64)
    neg_i = neg_samples.astype(jnp.int32).reshape(B * NEG // 64, 64)
    out_w, out_c, out_n = _run(ctx_i, cen_i, neg_i, center_table, context_table)
    return (out_w, out_c, out_n.reshape(B, NEG, DIM))
